# Initial kernel scaffold; baseline (speedup 1.0000x reference)
import jax
import jax.numpy as jnp
from jax.experimental import pallas as pl


def kernel(edge_index, state, emb, W1, G1, B1, W2, GBN, BBN, Wr1, br1, Wr2, br2):
    raise NotImplementedError("write your pallas kernel here")


# speedup vs baseline: 4.8562x; 4.8562x over previous
import functools

import jax
import jax.numpy as jnp
from jax import lax
from jax.experimental import pallas as pl
from jax.experimental.pallas import tpu as pltpu
from jax.experimental.pallas import tpu_sc as plsc

N = 10000
E = 320000
H = 128
L = 4

_NC = 2
_NS = 16
_NW = _NC * _NS
_EPW = E // _NW
_C = 80
_NCHUNK = _EPW // _C
_NP = 10240
_RPT = _NP // _NS
_ZROWS = 128


def _sc_segment_sum(h, src, dst):
    mesh = plsc.VectorSubcoreMesh(core_axis_name="c", subcore_axis_name="s")

    @functools.partial(
        pl.kernel,
        out_type=jax.ShapeDtypeStruct((2 * _NP, H), jnp.float32),
        mesh=mesh,
        scratch_types=[
            pltpu.VMEM((_C,), jnp.int32),
            pltpu.VMEM((_C,), jnp.int32),
            pltpu.VMEM((_C, H), jnp.float32),
            pltpu.VMEM((_ZROWS, H), jnp.float32),
            pltpu.VMEM_SHARED((_NP, H), jnp.float32),
            pltpu.SemaphoreType.DMA,
        ],
    )
    def k(h_hbm, src_hbm, dst_hbm, out_hbm, src_v, dst_v, rows_v, zb_v,
          agg_sh, sem):
        cid = lax.axis_index("c")
        sid = lax.axis_index("s")
        wid = cid * _NS + sid

        z16 = jnp.zeros((16,), jnp.float32)

        def zrow(i, carry):
            for j in range(H // 16):
                zb_v[i, pl.ds(j * 16, 16)] = z16
            return carry

        lax.fori_loop(0, _ZROWS, zrow, 0)
        for t in range(_RPT // _ZROWS):
            pltpu.sync_copy(
                zb_v, agg_sh.at[pl.ds(sid * _RPT + t * _ZROWS, _ZROWS)])
        plsc.subcore_barrier()

        base = wid * _EPW

        def body(j, carry):
            off = base + j * _C
            pltpu.sync_copy(src_hbm.at[pl.ds(off, _C)], src_v)
            pltpu.sync_copy(dst_hbm.at[pl.ds(off, _C)], dst_v)
            pltpu.async_copy(h_hbm.at[src_v], rows_v, sem).wait()
            pltpu.sync_copy(rows_v, agg_sh.at[dst_v], add=True)
            return carry

        lax.fori_loop(0, _NCHUNK, body, 0)
        plsc.subcore_barrier()

        pltpu.sync_copy(
            agg_sh.at[pl.ds(sid * _RPT, _RPT)],
            out_hbm.at[pl.ds(cid * _NP + sid * _RPT, _RPT)])

    return k(h, src, dst)


def _embed_body(state_ref, emb_ref, out_ref):
    s = state_ref[...]
    e0 = emb_ref[0:1, :]
    e1 = emb_ref[1:2, :]
    out_ref[...] = jnp.where(s == 0, e0, e1)


def _tc_embed(state, emb):
    return pl.pallas_call(
        _embed_body,
        out_shape=jax.ShapeDtypeStruct((N, H), jnp.float32),
    )(state.reshape(N, 1), emb)


def _bn(z, g, b):
    m = jnp.mean(z, axis=0, keepdims=True)
    v = jnp.mean((z - m) ** 2, axis=0, keepdims=True)
    return (z - m) * lax.rsqrt(v + 1e-5) * g + b


def _layer_body(h_ref, p_ref, w1_ref, g1_ref, b1_ref, w2_ref, gbn_ref,
                bbn_ref, out_ref):
    rst = h_ref[...] + p_ref[0:N, :] + p_ref[_NP:_NP + N, :]
    z = jnp.dot(rst, w1_ref[...], preferred_element_type=jnp.float32)
    z = _bn(z, g1_ref[...], b1_ref[...])
    z = jnp.maximum(z, 0.0)
    z = jnp.dot(z, w2_ref[...], preferred_element_type=jnp.float32)
    z = _bn(z, gbn_ref[...], bbn_ref[...])
    out_ref[...] = jnp.maximum(z, 0.0)


def _tc_layer(h, parts, w1, g1, b1, w2, gbn, bbn):
    return pl.pallas_call(
        _layer_body,
        out_shape=jax.ShapeDtypeStruct((N, H), jnp.float32),
    )(h, parts, w1, g1.reshape(1, H), b1.reshape(1, H), w2,
      gbn.reshape(1, H), bbn.reshape(1, H))


def _readout_body(h0, h1, h2, h3, h4, wr1, br1, wr2, br2, out_ref):
    acc = jnp.dot(h0[...], wr1[0:H, :], preferred_element_type=jnp.float32)
    acc += jnp.dot(h1[...], wr1[H:2 * H, :], preferred_element_type=jnp.float32)
    acc += jnp.dot(h2[...], wr1[2 * H:3 * H, :], preferred_element_type=jnp.float32)
    acc += jnp.dot(h3[...], wr1[3 * H:4 * H, :], preferred_element_type=jnp.float32)
    acc += jnp.dot(h4[...], wr1[4 * H:5 * H, :], preferred_element_type=jnp.float32)
    hr = jnp.maximum(acc + br1[...], 0.0)
    out_ref[...] = jnp.dot(hr, wr2[...], preferred_element_type=jnp.float32) + br2[...]


def _tc_readout(hs, wr1, br1, wr2, br2):
    return pl.pallas_call(
        _readout_body,
        out_shape=jax.ShapeDtypeStruct((N, 1), jnp.float32),
    )(*hs, wr1, br1.reshape(1, H), wr2, br2.reshape(1, 1))


def kernel(edge_index, state, emb, W1, G1, B1, W2, GBN, BBN, Wr1, br1, Wr2,
           br2):
    src = edge_index[0]
    dst = edge_index[1]
    h = _tc_embed(state, emb)
    hs = [h]
    for i in range(L):
        parts = _sc_segment_sum(h, src, dst)
        h = _tc_layer(h, parts, W1[i], G1[i], B1[i], W2[i], GBN[i], BBN[i])
        hs.append(h)
    return _tc_readout(hs, Wr1, br1, Wr2, br2)
